# Initial kernel scaffold; baseline (speedup 1.0000x reference)
import jax
import jax.numpy as jnp
from jax.experimental import pallas as pl


def kernel(hidden_states, w_gate, w1, b1, w2, b2):
    raise NotImplementedError("write your pallas kernel here")


# speedup vs baseline: 1.2035x; 1.2035x over previous
import functools

import jax
import jax.numpy as jnp
from jax import lax
from jax.experimental import pallas as pl
from jax.experimental.pallas import tpu as pltpu
from jax.experimental.pallas import tpu_sc as plsc

E = 64
D = 768
F = 3072
T = 2048
CAP = 128
LAM = 0.01
TRASH = E * CAP
NROWS = (E + 1) * CAP
TB = 256
NTB = T // TB
FCH = 768
NFC = F // FCH
NW = 32
RPW = T // NW


def _gate_body(x_ref, wg_ref, slot_ref, loss_ref, off_ref, acc_ref):
    i = pl.program_id(0)

    @pl.when(i == 0)
    def _():
        off_ref[...] = jnp.zeros_like(off_ref)
        acc_ref[...] = jnp.zeros_like(acc_ref)

    x = x_ref[...]
    logits = lax.dot_general(x, wg_ref[...], (((1,), (1,)), ((), ())),
                             preferred_element_type=jnp.float32)
    mx = jnp.max(logits, axis=1, keepdims=True)
    p = jnp.exp(logits - mx)
    probs = p / jnp.sum(p, axis=1, keepdims=True)

    col = lax.broadcasted_iota(jnp.int32, (TB, E), 1)
    eid = jnp.min(jnp.where(logits == mx, col, E), axis=1, keepdims=True)
    onehot = (col == eid).astype(jnp.float32)

    r = lax.broadcasted_iota(jnp.int32, (TB, TB), 0)
    c = lax.broadcasted_iota(jnp.int32, (TB, TB), 1)
    ltri = (r >= c).astype(jnp.float32)
    csum = lax.dot_general(ltri, onehot, (((1,), (0,)), ((), ())),
                           preferred_element_type=jnp.float32)
    pos_incl = jnp.sum(csum * onehot, axis=1, keepdims=True)
    off_tok = jnp.sum(off_ref[...] * onehot, axis=1, keepdims=True)
    pos = off_tok + pos_incl - 1.0
    slot = eid.astype(jnp.float32) * CAP + pos
    slot = jnp.where(pos < CAP, slot, float(TRASH))
    slot_ref[...] = slot.astype(jnp.int32)[None]

    off_ref[...] = off_ref[...] + jnp.sum(onehot, axis=0, keepdims=True)
    acc_ref[...] = acc_ref[...] + jnp.sum(probs, axis=0, keepdims=True)

    @pl.when(i == NTB - 1)
    def _():
        f = off_ref[...] / float(T)
        var_f = jnp.sum((f - jnp.mean(f)) ** 2) / (E - 1)
        im = acc_ref[...] / float(T)
        var_i = jnp.sum((im - jnp.mean(im)) ** 2) / (E - 1)
        loss_ref[...] = jnp.broadcast_to(LAM * (var_f + var_i), (1, 1))


def _gate(x, wg):
    return pl.pallas_call(
        _gate_body,
        grid=(NTB,),
        in_specs=[pl.BlockSpec((TB, D), lambda i: (i, 0)),
                  pl.BlockSpec((E, D), lambda i: (0, 0))],
        out_specs=[pl.BlockSpec((1, TB, 1), lambda i: (i, 0, 0)),
                   pl.BlockSpec((1, 1), lambda i: (0, 0))],
        out_shape=[jax.ShapeDtypeStruct((NTB, TB, 1), jnp.int32),
                   jax.ShapeDtypeStruct((1, 1), jnp.float32)],
        scratch_shapes=[pltpu.VMEM((1, E), jnp.float32),
                        pltpu.VMEM((1, E), jnp.float32)],
    )(x, wg)


def _ffn_body(disp_ref, w1_ref, b1_ref, w2_ref, b2_ref, y_ref):
    e = pl.program_id(0)
    j = pl.program_id(1)
    d = disp_ref[...]
    h = lax.dot_general(d, w1_ref[0], (((1,), (1,)), ((), ())),
                        preferred_element_type=jnp.float32)
    h = h + b1_ref[0]
    h = 0.5 * h * (1.0 + lax.erf(h * 0.7071067811865476))
    part = lax.dot_general(h, w2_ref[0], (((1,), (1,)), ((), ())),
                           preferred_element_type=jnp.float32)

    @pl.when(j == 0)
    def _():
        y_ref[...] = part

    @pl.when(j > 0)
    def _():
        y_ref[...] = y_ref[...] + part

    @pl.when(j == NFC - 1)
    def _():
        y_ref[...] = y_ref[...] + b2_ref[0]

    @pl.when(e == E)
    def _():
        y_ref[...] = jnp.zeros_like(y_ref)


def _ffn(disp, w1, b1r, w2, b2r):
    ce = lambda e: jnp.minimum(e, E - 1)
    return pl.pallas_call(
        _ffn_body,
        grid=(E + 1, NFC),
        in_specs=[
            pl.BlockSpec((CAP, D), lambda e, j: (e, 0)),
            pl.BlockSpec((1, FCH, D), lambda e, j: (ce(e), j, 0)),
            pl.BlockSpec((1, 1, FCH), lambda e, j: (ce(e) * NFC + j, 0, 0)),
            pl.BlockSpec((1, D, FCH), lambda e, j: (ce(e), 0, j)),
            pl.BlockSpec((1, 1, D), lambda e, j: (ce(e), 0, 0)),
        ],
        out_specs=pl.BlockSpec((CAP, D), lambda e, j: (e, 0)),
        out_shape=jax.ShapeDtypeStruct((NROWS, D), jnp.float32),
        compiler_params=pltpu.CompilerParams(
            dimension_semantics=("arbitrary", "arbitrary")),
    )(disp, w1, b1r, w2, b2r)


def _wid():
    return lax.axis_index("s") * 2 + lax.axis_index("c")


def _dispatch_body(x_hbm, slot_hbm, disp_hbm, idx_v, rows_v, sem):
    base = _wid() * RPW
    pltpu.sync_copy(slot_hbm.at[pl.ds(base, RPW)], idx_v)
    pltpu.sync_copy(x_hbm.at[pl.ds(base, RPW)], rows_v)
    pltpu.async_copy(rows_v, disp_hbm.at[idx_v], sem).wait()


def _combine_body(y_hbm, slot_hbm, out_hbm, idx_v, rows_v, sem):
    base = _wid() * RPW
    pltpu.sync_copy(slot_hbm.at[pl.ds(base, RPW)], idx_v)
    pltpu.async_copy(y_hbm.at[idx_v], rows_v, sem).wait()
    pltpu.sync_copy(rows_v, out_hbm.at[pl.ds(base, RPW)])


@functools.cache
def _sc_calls():
    mesh = plsc.VectorSubcoreMesh(core_axis_name="c", subcore_axis_name="s")
    scratch = [pltpu.VMEM((RPW,), jnp.int32),
               pltpu.VMEM((RPW, D), jnp.float32),
               pltpu.SemaphoreType.DMA]
    dispatch = pl.kernel(
        _dispatch_body,
        out_type=jax.ShapeDtypeStruct((NROWS, D), jnp.float32),
        mesh=mesh, scratch_types=scratch)
    combine = pl.kernel(
        _combine_body,
        out_type=jax.ShapeDtypeStruct((T, D), jnp.float32),
        mesh=mesh, scratch_types=scratch)
    return dispatch, combine


def kernel(hidden_states, w_gate, w1, b1, w2, b2):
    B, S, C = hidden_states.shape
    x = hidden_states.reshape(T, D)
    slot3, loss = _gate(x, w_gate)
    slot = slot3.reshape(T)
    dispatch, combine = _sc_calls()
    disp = dispatch(x, slot)
    y = _ffn(disp, w1, b1.reshape(E * NFC, 1, FCH), w2, b2.reshape(E, 1, D))
    out = combine(y, slot)
    return out.reshape(B, S, C), loss[0, 0]


# speedup vs baseline: 1.4500x; 1.2048x over previous
import functools

import jax
import jax.numpy as jnp
from jax import lax
from jax.experimental import pallas as pl
from jax.experimental.pallas import tpu as pltpu
from jax.experimental.pallas import tpu_sc as plsc

E = 64
D = 768
F = 3072
T = 2048
CAP = 128
LAM = 0.01
TRASH = E * CAP
NROWS = (E + 1) * CAP
TB = 256
NTB = T // TB
FCH = 1536
NFC = F // FCH
NW = 32
RPW = T // NW


def _gate_body(x_ref, wg_ref, slot_ref, loss_ref, off_ref, acc_ref):
    i = pl.program_id(0)

    @pl.when(i == 0)
    def _():
        off_ref[...] = jnp.zeros_like(off_ref)
        acc_ref[...] = jnp.zeros_like(acc_ref)

    x = x_ref[...]
    logits = lax.dot_general(x, wg_ref[...], (((1,), (1,)), ((), ())),
                             preferred_element_type=jnp.float32)
    mx = jnp.max(logits, axis=1, keepdims=True)
    p = jnp.exp(logits - mx)
    probs = p / jnp.sum(p, axis=1, keepdims=True)

    col = lax.broadcasted_iota(jnp.int32, (TB, E), 1)
    eid = jnp.min(jnp.where(logits == mx, col, E), axis=1, keepdims=True)
    onehot = (col == eid).astype(jnp.float32)

    r = lax.broadcasted_iota(jnp.int32, (TB, TB), 0)
    c = lax.broadcasted_iota(jnp.int32, (TB, TB), 1)
    ltri = (r >= c).astype(jnp.float32)
    csum = lax.dot_general(ltri, onehot, (((1,), (0,)), ((), ())),
                           preferred_element_type=jnp.float32)
    pos_incl = jnp.sum(csum * onehot, axis=1, keepdims=True)
    off_tok = jnp.sum(off_ref[...] * onehot, axis=1, keepdims=True)
    pos = off_tok + pos_incl - 1.0
    slot = eid.astype(jnp.float32) * CAP + pos
    slot = jnp.where(pos < CAP, slot, float(TRASH))
    slot_ref[...] = slot.astype(jnp.int32)[None]

    off_ref[...] = off_ref[...] + jnp.sum(onehot, axis=0, keepdims=True)
    acc_ref[...] = acc_ref[...] + jnp.sum(probs, axis=0, keepdims=True)

    @pl.when(i == NTB - 1)
    def _():
        f = off_ref[...] / float(T)
        var_f = jnp.sum((f - jnp.mean(f)) ** 2) / (E - 1)
        im = acc_ref[...] / float(T)
        var_i = jnp.sum((im - jnp.mean(im)) ** 2) / (E - 1)
        loss_ref[...] = jnp.broadcast_to(LAM * (var_f + var_i), (1, 1))


def _gate(x, wg):
    return pl.pallas_call(
        _gate_body,
        grid=(NTB,),
        in_specs=[pl.BlockSpec((TB, D), lambda i: (i, 0)),
                  pl.BlockSpec((E, D), lambda i: (0, 0))],
        out_specs=[pl.BlockSpec((1, TB, 1), lambda i: (i, 0, 0)),
                   pl.BlockSpec((1, 1), lambda i: (0, 0))],
        out_shape=[jax.ShapeDtypeStruct((NTB, TB, 1), jnp.int32),
                   jax.ShapeDtypeStruct((1, 1), jnp.float32)],
        scratch_shapes=[pltpu.VMEM((1, E), jnp.float32),
                        pltpu.VMEM((1, E), jnp.float32)],
    )(x, wg)


def _ffn_body(disp_ref, w1_ref, b1_ref, w2_ref, b2_ref, y_ref):
    e = pl.program_id(0)
    j = pl.program_id(1)
    d = disp_ref[...]
    h = lax.dot_general(d, w1_ref[0], (((1,), (1,)), ((), ())),
                        preferred_element_type=jnp.float32)
    h = h + b1_ref[0]
    h = 0.5 * h * (1.0 + lax.erf(h * 0.7071067811865476))
    part = lax.dot_general(h, w2_ref[0], (((1,), (1,)), ((), ())),
                           preferred_element_type=jnp.float32)

    @pl.when(j == 0)
    def _():
        y_ref[...] = part

    @pl.when(j > 0)
    def _():
        y_ref[...] = y_ref[...] + part

    @pl.when(j == NFC - 1)
    def _():
        y_ref[...] = y_ref[...] + b2_ref[0]

    @pl.when(e == E)
    def _():
        y_ref[...] = jnp.zeros_like(y_ref)


def _ffn(disp, w1, b1r, w2, b2r):
    ce = lambda e: jnp.minimum(e, E - 1)
    return pl.pallas_call(
        _ffn_body,
        grid=(E + 1, NFC),
        in_specs=[
            pl.BlockSpec((CAP, D), lambda e, j: (e, 0)),
            pl.BlockSpec((1, FCH, D), lambda e, j: (ce(e), j, 0)),
            pl.BlockSpec((1, 1, FCH), lambda e, j: (ce(e) * NFC + j, 0, 0)),
            pl.BlockSpec((1, D, FCH), lambda e, j: (ce(e), 0, j)),
            pl.BlockSpec((1, 1, D), lambda e, j: (ce(e), 0, 0)),
        ],
        out_specs=pl.BlockSpec((CAP, D), lambda e, j: (e, 0)),
        out_shape=jax.ShapeDtypeStruct((NROWS, D), jnp.float32),
        compiler_params=pltpu.CompilerParams(
            dimension_semantics=("arbitrary", "arbitrary")),
    )(disp, w1, b1r, w2, b2r)


def _wid():
    return lax.axis_index("s") * 2 + lax.axis_index("c")


def _dispatch_body(x_hbm, slot_hbm, disp_hbm, idx_v, rows_v, sem):
    base = _wid() * RPW
    pltpu.sync_copy(slot_hbm.at[pl.ds(base, RPW)], idx_v)
    pltpu.sync_copy(x_hbm.at[pl.ds(base, RPW)], rows_v)
    pltpu.async_copy(rows_v, disp_hbm.at[idx_v], sem).wait()


def _combine_body(y_hbm, slot_hbm, out_hbm, idx_v, rows_v, sem):
    base = _wid() * RPW
    pltpu.sync_copy(slot_hbm.at[pl.ds(base, RPW)], idx_v)
    pltpu.async_copy(y_hbm.at[idx_v], rows_v, sem).wait()
    pltpu.sync_copy(rows_v, out_hbm.at[pl.ds(base, RPW)])


@functools.cache
def _sc_calls():
    mesh = plsc.VectorSubcoreMesh(core_axis_name="c", subcore_axis_name="s")
    scratch = [pltpu.VMEM((RPW,), jnp.int32),
               pltpu.VMEM((RPW, D), jnp.float32),
               pltpu.SemaphoreType.DMA]
    dispatch = pl.kernel(
        _dispatch_body,
        out_type=jax.ShapeDtypeStruct((NROWS, D), jnp.float32),
        mesh=mesh, scratch_types=scratch)
    combine = pl.kernel(
        _combine_body,
        out_type=jax.ShapeDtypeStruct((T, D), jnp.float32),
        mesh=mesh, scratch_types=scratch)
    return dispatch, combine


def kernel(hidden_states, w_gate, w1, b1, w2, b2):
    B, S, C = hidden_states.shape
    x = hidden_states.reshape(T, D)
    slot3, loss = _gate(x, w_gate)
    slot = slot3.reshape(T)
    dispatch, combine = _sc_calls()
    disp = dispatch(x, slot)
    y = _ffn(disp, w1, b1.reshape(E * NFC, 1, FCH), w2, b2.reshape(E, 1, D))
    out = combine(y, slot)
    return out.reshape(B, S, C), loss[0, 0]


# speedup vs baseline: 1.4992x; 1.0340x over previous
import functools

import jax
import jax.numpy as jnp
from jax import lax
from jax.experimental import pallas as pl
from jax.experimental.pallas import tpu as pltpu
from jax.experimental.pallas import tpu_sc as plsc

E = 64
D = 768
F = 3072
T = 2048
CAP = 128
LAM = 0.01
TRASH = E * CAP
NROWS = (E + 1) * CAP
TB = 256
NTB = T // TB
FCH = 3072
NFC = F // FCH
NW = 32
RPW = T // NW


def _gate_body(x_ref, wg_ref, slot_ref, loss_ref, off_ref, acc_ref):
    i = pl.program_id(0)

    @pl.when(i == 0)
    def _():
        off_ref[...] = jnp.zeros_like(off_ref)
        acc_ref[...] = jnp.zeros_like(acc_ref)

    x = x_ref[...]
    logits = lax.dot_general(x, wg_ref[...], (((1,), (1,)), ((), ())),
                             preferred_element_type=jnp.float32)
    mx = jnp.max(logits, axis=1, keepdims=True)
    p = jnp.exp(logits - mx)
    probs = p / jnp.sum(p, axis=1, keepdims=True)

    col = lax.broadcasted_iota(jnp.int32, (TB, E), 1)
    eid = jnp.min(jnp.where(logits == mx, col, E), axis=1, keepdims=True)
    onehot = (col == eid).astype(jnp.float32)

    r = lax.broadcasted_iota(jnp.int32, (TB, TB), 0)
    c = lax.broadcasted_iota(jnp.int32, (TB, TB), 1)
    ltri = (r >= c).astype(jnp.float32)
    csum = lax.dot_general(ltri, onehot, (((1,), (0,)), ((), ())),
                           preferred_element_type=jnp.float32)
    pos_incl = jnp.sum(csum * onehot, axis=1, keepdims=True)
    off_tok = jnp.sum(off_ref[...] * onehot, axis=1, keepdims=True)
    pos = off_tok + pos_incl - 1.0
    slot = eid.astype(jnp.float32) * CAP + pos
    slot = jnp.where(pos < CAP, slot, float(TRASH))
    slot_ref[...] = slot.astype(jnp.int32)[None]

    off_ref[...] = off_ref[...] + jnp.sum(onehot, axis=0, keepdims=True)
    acc_ref[...] = acc_ref[...] + jnp.sum(probs, axis=0, keepdims=True)

    @pl.when(i == NTB - 1)
    def _():
        f = off_ref[...] / float(T)
        var_f = jnp.sum((f - jnp.mean(f)) ** 2) / (E - 1)
        im = acc_ref[...] / float(T)
        var_i = jnp.sum((im - jnp.mean(im)) ** 2) / (E - 1)
        loss_ref[...] = jnp.broadcast_to(LAM * (var_f + var_i), (1, 1))


def _gate(x, wg):
    return pl.pallas_call(
        _gate_body,
        grid=(NTB,),
        in_specs=[pl.BlockSpec((TB, D), lambda i: (i, 0)),
                  pl.BlockSpec((E, D), lambda i: (0, 0))],
        out_specs=[pl.BlockSpec((1, TB, 1), lambda i: (i, 0, 0)),
                   pl.BlockSpec((1, 1), lambda i: (0, 0))],
        out_shape=[jax.ShapeDtypeStruct((NTB, TB, 1), jnp.int32),
                   jax.ShapeDtypeStruct((1, 1), jnp.float32)],
        scratch_shapes=[pltpu.VMEM((1, E), jnp.float32),
                        pltpu.VMEM((1, E), jnp.float32)],
    )(x, wg)


def _ffn_body(disp_ref, w1_ref, b1_ref, w2_ref, b2_ref, y_ref):
    e = pl.program_id(0)
    j = pl.program_id(1)
    d = disp_ref[...]
    h = lax.dot_general(d, w1_ref[0], (((1,), (1,)), ((), ())),
                        preferred_element_type=jnp.float32)
    h = h + b1_ref[0]
    h = 0.5 * h * (1.0 + lax.erf(h * 0.7071067811865476))
    part = lax.dot_general(h, w2_ref[0], (((1,), (1,)), ((), ())),
                           preferred_element_type=jnp.float32)

    @pl.when(j == 0)
    def _():
        y_ref[...] = part

    @pl.when(j > 0)
    def _():
        y_ref[...] = y_ref[...] + part

    @pl.when(j == NFC - 1)
    def _():
        y_ref[...] = y_ref[...] + b2_ref[0]

    @pl.when(e == E)
    def _():
        y_ref[...] = jnp.zeros_like(y_ref)


def _ffn(disp, w1, b1r, w2, b2r):
    ce = lambda e: jnp.minimum(e, E - 1)
    return pl.pallas_call(
        _ffn_body,
        grid=(E + 1, NFC),
        in_specs=[
            pl.BlockSpec((CAP, D), lambda e, j: (e, 0)),
            pl.BlockSpec((1, FCH, D), lambda e, j: (ce(e), j, 0)),
            pl.BlockSpec((1, 1, FCH), lambda e, j: (ce(e) * NFC + j, 0, 0)),
            pl.BlockSpec((1, D, FCH), lambda e, j: (ce(e), 0, j)),
            pl.BlockSpec((1, 1, D), lambda e, j: (ce(e), 0, 0)),
        ],
        out_specs=pl.BlockSpec((CAP, D), lambda e, j: (e, 0)),
        out_shape=jax.ShapeDtypeStruct((NROWS, D), jnp.float32),
        compiler_params=pltpu.CompilerParams(
            dimension_semantics=("arbitrary", "arbitrary")),
    )(disp, w1, b1r, w2, b2r)


def _wid():
    return lax.axis_index("s") * 2 + lax.axis_index("c")


def _dispatch_body(x_hbm, slot_hbm, disp_hbm, idx_v, rows_v, sem):
    base = _wid() * RPW
    pltpu.sync_copy(slot_hbm.at[pl.ds(base, RPW)], idx_v)
    pltpu.sync_copy(x_hbm.at[pl.ds(base, RPW)], rows_v)
    pltpu.async_copy(rows_v, disp_hbm.at[idx_v], sem).wait()


def _combine_body(y_hbm, slot_hbm, out_hbm, idx_v, rows_v, sem):
    base = _wid() * RPW
    pltpu.sync_copy(slot_hbm.at[pl.ds(base, RPW)], idx_v)
    pltpu.async_copy(y_hbm.at[idx_v], rows_v, sem).wait()
    pltpu.sync_copy(rows_v, out_hbm.at[pl.ds(base, RPW)])


@functools.cache
def _sc_calls():
    mesh = plsc.VectorSubcoreMesh(core_axis_name="c", subcore_axis_name="s")
    scratch = [pltpu.VMEM((RPW,), jnp.int32),
               pltpu.VMEM((RPW, D), jnp.float32),
               pltpu.SemaphoreType.DMA]
    dispatch = pl.kernel(
        _dispatch_body,
        out_type=jax.ShapeDtypeStruct((NROWS, D), jnp.float32),
        mesh=mesh, scratch_types=scratch)
    combine = pl.kernel(
        _combine_body,
        out_type=jax.ShapeDtypeStruct((T, D), jnp.float32),
        mesh=mesh, scratch_types=scratch)
    return dispatch, combine


def kernel(hidden_states, w_gate, w1, b1, w2, b2):
    B, S, C = hidden_states.shape
    x = hidden_states.reshape(T, D)
    slot3, loss = _gate(x, w_gate)
    slot = slot3.reshape(T)
    dispatch, combine = _sc_calls()
    disp = dispatch(x, slot)
    y = _ffn(disp, w1, b1.reshape(E * NFC, 1, FCH), w2, b2.reshape(E, 1, D))
    out = combine(y, slot)
    return out.reshape(B, S, C), loss[0, 0]


# speedup vs baseline: 1.4996x; 1.0003x over previous
import functools

import jax
import jax.numpy as jnp
from jax import lax
from jax.experimental import pallas as pl
from jax.experimental.pallas import tpu as pltpu
from jax.experimental.pallas import tpu_sc as plsc

E = 64
D = 768
F = 3072
T = 2048
CAP = 128
LAM = 0.01
TRASH = E * CAP
NROWS = (E + 1) * CAP
TB = 256
NTB = T // TB
FCH = 3072
NFC = F // FCH
NW = 32
RPW = T // NW


def _gate_body(x_ref, wg_ref, slot_ref, loss_ref, off_ref, acc_ref):
    i = pl.program_id(0)

    @pl.when(i == 0)
    def _():
        off_ref[...] = jnp.zeros_like(off_ref)
        acc_ref[...] = jnp.zeros_like(acc_ref)

    x = x_ref[...]
    logits = lax.dot_general(x, wg_ref[...], (((1,), (1,)), ((), ())),
                             preferred_element_type=jnp.float32)
    mx = jnp.max(logits, axis=1, keepdims=True)
    p = jnp.exp(logits - mx)
    probs = p / jnp.sum(p, axis=1, keepdims=True)

    col = lax.broadcasted_iota(jnp.int32, (TB, E), 1)
    eid = jnp.min(jnp.where(logits == mx, col, E), axis=1, keepdims=True)
    onehot = (col == eid).astype(jnp.float32)

    r = lax.broadcasted_iota(jnp.int32, (TB, TB), 0)
    c = lax.broadcasted_iota(jnp.int32, (TB, TB), 1)
    ltri = (r >= c).astype(jnp.float32)
    csum = lax.dot_general(ltri, onehot, (((1,), (0,)), ((), ())),
                           preferred_element_type=jnp.float32)
    pos_incl = jnp.sum(csum * onehot, axis=1, keepdims=True)
    off_tok = jnp.sum(off_ref[...] * onehot, axis=1, keepdims=True)
    pos = off_tok + pos_incl - 1.0
    slot = eid.astype(jnp.float32) * CAP + pos
    slot = jnp.where(pos < CAP, slot, float(TRASH))
    slot_ref[...] = slot.astype(jnp.int32)[None]

    off_ref[...] = off_ref[...] + jnp.sum(onehot, axis=0, keepdims=True)
    acc_ref[...] = acc_ref[...] + jnp.sum(probs, axis=0, keepdims=True)

    @pl.when(i == NTB - 1)
    def _():
        f = off_ref[...] / float(T)
        var_f = jnp.sum((f - jnp.mean(f)) ** 2) / (E - 1)
        im = acc_ref[...] / float(T)
        var_i = jnp.sum((im - jnp.mean(im)) ** 2) / (E - 1)
        loss_ref[...] = jnp.broadcast_to(LAM * (var_f + var_i), (1, 1))


def _gate(x, wg):
    return pl.pallas_call(
        _gate_body,
        grid=(NTB,),
        in_specs=[pl.BlockSpec((TB, D), lambda i: (i, 0)),
                  pl.BlockSpec((E, D), lambda i: (0, 0))],
        out_specs=[pl.BlockSpec((1, TB, 1), lambda i: (i, 0, 0)),
                   pl.BlockSpec((1, 1), lambda i: (0, 0))],
        out_shape=[jax.ShapeDtypeStruct((NTB, TB, 1), jnp.int32),
                   jax.ShapeDtypeStruct((1, 1), jnp.float32)],
        scratch_shapes=[pltpu.VMEM((1, E), jnp.float32),
                        pltpu.VMEM((1, E), jnp.float32)],
    )(x, wg)


def _ffn_body(disp_ref, w1_ref, b1_ref, w2_ref, b2_ref, y_ref):
    e = pl.program_id(0)
    j = pl.program_id(1)
    d = disp_ref[...]
    h = lax.dot_general(d, w1_ref[0], (((1,), (1,)), ((), ())),
                        preferred_element_type=jnp.float32)
    h = h + b1_ref[0]
    h = 0.5 * h * (1.0 + lax.erf(h * 0.7071067811865476))
    part = lax.dot_general(h, w2_ref[0], (((1,), (1,)), ((), ())),
                           preferred_element_type=jnp.float32)

    @pl.when(j == 0)
    def _():
        y_ref[...] = part

    @pl.when(j > 0)
    def _():
        y_ref[...] = y_ref[...] + part

    @pl.when(j == NFC - 1)
    def _():
        y_ref[...] = y_ref[...] + b2_ref[0]

    @pl.when(e == E)
    def _():
        y_ref[...] = jnp.zeros_like(y_ref)


def _ffn(disp, w1, b1r, w2, b2r):
    ce = lambda e: jnp.minimum(e, E - 1)
    return pl.pallas_call(
        _ffn_body,
        grid=(E + 1, NFC),
        in_specs=[
            pl.BlockSpec((CAP, D), lambda e, j: (e, 0)),
            pl.BlockSpec((1, FCH, D), lambda e, j: (ce(e), j, 0)),
            pl.BlockSpec((1, 1, FCH), lambda e, j: (ce(e) * NFC + j, 0, 0)),
            pl.BlockSpec((1, D, FCH), lambda e, j: (ce(e), 0, j)),
            pl.BlockSpec((1, 1, D), lambda e, j: (ce(e), 0, 0)),
        ],
        out_specs=pl.BlockSpec((CAP, D), lambda e, j: (e, 0)),
        out_shape=jax.ShapeDtypeStruct((NROWS, D), jnp.float32),
        compiler_params=pltpu.CompilerParams(
            dimension_semantics=("arbitrary", "arbitrary")),
    )(disp, w1, b1r, w2, b2r)


def _wid():
    return lax.axis_index("s") * 2 + lax.axis_index("c")


HW = RPW // 2


def _dispatch_body(x_hbm, slot_hbm, disp_hbm, idx_v, rows_v, sem_a, sem_b, sem_i):
    w = _wid()
    base = w * RPW
    ci = pltpu.async_copy(slot_hbm.at[w], idx_v, sem_i)
    c0 = pltpu.async_copy(x_hbm.at[pl.ds(base, HW)], rows_v.at[pl.ds(0, HW)], sem_a)
    c1 = pltpu.async_copy(x_hbm.at[pl.ds(base + HW, HW)], rows_v.at[pl.ds(HW, HW)], sem_b)
    ci.wait()
    c0.wait()
    s0 = pltpu.async_copy(rows_v.at[pl.ds(0, HW)], disp_hbm.at[idx_v.at[0]], sem_a)
    c1.wait()
    s1 = pltpu.async_copy(rows_v.at[pl.ds(HW, HW)], disp_hbm.at[idx_v.at[1]], sem_b)
    s0.wait()
    s1.wait()


def _combine_body(y_hbm, slot_hbm, out_hbm, idx_v, rows_v, sem_a, sem_b, sem_i):
    w = _wid()
    base = w * RPW
    pltpu.async_copy(slot_hbm.at[w], idx_v, sem_i).wait()
    g0 = pltpu.async_copy(y_hbm.at[idx_v.at[0]], rows_v.at[pl.ds(0, HW)], sem_a)
    g1 = pltpu.async_copy(y_hbm.at[idx_v.at[1]], rows_v.at[pl.ds(HW, HW)], sem_b)
    g0.wait()
    w0 = pltpu.async_copy(rows_v.at[pl.ds(0, HW)], out_hbm.at[pl.ds(base, HW)], sem_a)
    g1.wait()
    w1 = pltpu.async_copy(rows_v.at[pl.ds(HW, HW)], out_hbm.at[pl.ds(base + HW, HW)], sem_b)
    w0.wait()
    w1.wait()


@functools.cache
def _sc_calls():
    mesh = plsc.VectorSubcoreMesh(core_axis_name="c", subcore_axis_name="s")
    scratch = [pltpu.VMEM((2, HW), jnp.int32),
               pltpu.VMEM((RPW, D), jnp.float32),
               pltpu.SemaphoreType.DMA,
               pltpu.SemaphoreType.DMA,
               pltpu.SemaphoreType.DMA]
    dispatch = pl.kernel(
        _dispatch_body,
        out_type=jax.ShapeDtypeStruct((NROWS, D), jnp.float32),
        mesh=mesh, scratch_types=scratch)
    combine = pl.kernel(
        _combine_body,
        out_type=jax.ShapeDtypeStruct((T, D), jnp.float32),
        mesh=mesh, scratch_types=scratch)
    return dispatch, combine


def kernel(hidden_states, w_gate, w1, b1, w2, b2):
    B, S, C = hidden_states.shape
    x = hidden_states.reshape(T, D)
    slot3, loss = _gate(x, w_gate)
    slot = slot3.reshape(NW, 2, HW)
    dispatch, combine = _sc_calls()
    disp = dispatch(x, slot)
    y = _ffn(disp, w1, b1.reshape(E * NFC, 1, FCH), w2, b2.reshape(E, 1, D))
    out = combine(y, slot)
    return out.reshape(B, S, C), loss[0, 0]


# speedup vs baseline: 1.5154x; 1.0105x over previous
import functools

import jax
import jax.numpy as jnp
from jax import lax
from jax.experimental import pallas as pl
from jax.experimental.pallas import tpu as pltpu
from jax.experimental.pallas import tpu_sc as plsc

E = 64
D = 768
F = 3072
T = 2048
CAP = 128
LAM = 0.01
TRASH = E * CAP
NROWS = (E + 1) * CAP
TB = 2048
NTB = T // TB
FCH = 3072
NFC = F // FCH
NW = 32
RPW = T // NW


def _gate_body(x_ref, wg_ref, slot_ref, loss_ref, off_ref, acc_ref):
    i = pl.program_id(0)

    @pl.when(i == 0)
    def _():
        off_ref[...] = jnp.zeros_like(off_ref)
        acc_ref[...] = jnp.zeros_like(acc_ref)

    x = x_ref[...]
    logits = lax.dot_general(x, wg_ref[...], (((1,), (1,)), ((), ())),
                             preferred_element_type=jnp.float32)
    mx = jnp.max(logits, axis=1, keepdims=True)
    p = jnp.exp(logits - mx)
    probs = p / jnp.sum(p, axis=1, keepdims=True)

    col = lax.broadcasted_iota(jnp.int32, (TB, E), 1)
    eid = jnp.min(jnp.where(logits == mx, col, E), axis=1, keepdims=True)
    onehot = (col == eid).astype(jnp.float32)

    r = lax.broadcasted_iota(jnp.int32, (TB, TB), 0)
    c = lax.broadcasted_iota(jnp.int32, (TB, TB), 1)
    ltri = (r >= c).astype(jnp.bfloat16)
    csum = lax.dot_general(ltri, onehot.astype(jnp.bfloat16),
                           (((1,), (0,)), ((), ())),
                           preferred_element_type=jnp.float32)
    pos_incl = jnp.sum(csum * onehot, axis=1, keepdims=True)
    off_tok = jnp.sum(off_ref[...] * onehot, axis=1, keepdims=True)
    pos = off_tok + pos_incl - 1.0
    slot = eid.astype(jnp.float32) * CAP + pos
    slot = jnp.where(pos < CAP, slot, float(TRASH))
    slot_ref[...] = slot.astype(jnp.int32)[None]

    off_ref[...] = off_ref[...] + jnp.sum(onehot, axis=0, keepdims=True)
    acc_ref[...] = acc_ref[...] + jnp.sum(probs, axis=0, keepdims=True)

    @pl.when(i == NTB - 1)
    def _():
        f = off_ref[...] / float(T)
        var_f = jnp.sum((f - jnp.mean(f)) ** 2) / (E - 1)
        im = acc_ref[...] / float(T)
        var_i = jnp.sum((im - jnp.mean(im)) ** 2) / (E - 1)
        loss_ref[...] = jnp.broadcast_to(LAM * (var_f + var_i), (1, 1))


def _gate(x, wg):
    return pl.pallas_call(
        _gate_body,
        grid=(NTB,),
        in_specs=[pl.BlockSpec((TB, D), lambda i: (i, 0)),
                  pl.BlockSpec((E, D), lambda i: (0, 0))],
        out_specs=[pl.BlockSpec((1, TB, 1), lambda i: (i, 0, 0)),
                   pl.BlockSpec((1, 1), lambda i: (0, 0))],
        out_shape=[jax.ShapeDtypeStruct((NTB, TB, 1), jnp.int32),
                   jax.ShapeDtypeStruct((1, 1), jnp.float32)],
        scratch_shapes=[pltpu.VMEM((1, E), jnp.float32),
                        pltpu.VMEM((1, E), jnp.float32)],
    )(x, wg)


def _ffn_body(disp_ref, w1_ref, b1_ref, w2_ref, b2_ref, y_ref):
    e = pl.program_id(0)
    j = pl.program_id(1)
    d = disp_ref[...]
    h = lax.dot_general(d, w1_ref[0], (((1,), (1,)), ((), ())),
                        preferred_element_type=jnp.float32)
    h = h + b1_ref[0]
    h = 0.5 * h * (1.0 + lax.erf(h * 0.7071067811865476))
    part = lax.dot_general(h, w2_ref[0], (((1,), (1,)), ((), ())),
                           preferred_element_type=jnp.float32)

    @pl.when(j == 0)
    def _():
        y_ref[...] = part

    @pl.when(j > 0)
    def _():
        y_ref[...] = y_ref[...] + part

    @pl.when(j == NFC - 1)
    def _():
        y_ref[...] = y_ref[...] + b2_ref[0]

    @pl.when(e == E)
    def _():
        y_ref[...] = jnp.zeros_like(y_ref)


def _ffn(disp, w1, b1r, w2, b2r):
    ce = lambda e: jnp.minimum(e, E - 1)
    return pl.pallas_call(
        _ffn_body,
        grid=(E + 1, NFC),
        in_specs=[
            pl.BlockSpec((CAP, D), lambda e, j: (e, 0)),
            pl.BlockSpec((1, FCH, D), lambda e, j: (ce(e), j, 0)),
            pl.BlockSpec((1, 1, FCH), lambda e, j: (ce(e) * NFC + j, 0, 0)),
            pl.BlockSpec((1, D, FCH), lambda e, j: (ce(e), 0, j)),
            pl.BlockSpec((1, 1, D), lambda e, j: (ce(e), 0, 0)),
        ],
        out_specs=pl.BlockSpec((CAP, D), lambda e, j: (e, 0)),
        out_shape=jax.ShapeDtypeStruct((NROWS, D), jnp.float32),
        compiler_params=pltpu.CompilerParams(
            dimension_semantics=("arbitrary", "arbitrary")),
    )(disp, w1, b1r, w2, b2r)


def _wid():
    return lax.axis_index("s") * 2 + lax.axis_index("c")


def _dispatch_body(x_hbm, slot_hbm, disp_hbm, idx_v, rows_v, sem):
    base = _wid() * RPW
    pltpu.sync_copy(slot_hbm.at[pl.ds(base, RPW)], idx_v)
    pltpu.sync_copy(x_hbm.at[pl.ds(base, RPW)], rows_v)
    pltpu.async_copy(rows_v, disp_hbm.at[idx_v], sem).wait()


def _combine_body(y_hbm, slot_hbm, out_hbm, idx_v, rows_v, sem):
    base = _wid() * RPW
    pltpu.sync_copy(slot_hbm.at[pl.ds(base, RPW)], idx_v)
    pltpu.async_copy(y_hbm.at[idx_v], rows_v, sem).wait()
    pltpu.sync_copy(rows_v, out_hbm.at[pl.ds(base, RPW)])


@functools.cache
def _sc_calls():
    mesh = plsc.VectorSubcoreMesh(core_axis_name="c", subcore_axis_name="s")
    scratch = [pltpu.VMEM((RPW,), jnp.int32),
               pltpu.VMEM((RPW, D), jnp.float32),
               pltpu.SemaphoreType.DMA]
    dispatch = pl.kernel(
        _dispatch_body,
        out_type=jax.ShapeDtypeStruct((NROWS, D), jnp.float32),
        mesh=mesh, scratch_types=scratch)
    combine = pl.kernel(
        _combine_body,
        out_type=jax.ShapeDtypeStruct((T, D), jnp.float32),
        mesh=mesh, scratch_types=scratch)
    return dispatch, combine


def kernel(hidden_states, w_gate, w1, b1, w2, b2):
    B, S, C = hidden_states.shape
    x = hidden_states.reshape(T, D)
    slot3, loss = _gate(x, w_gate)
    slot = slot3.reshape(T)
    dispatch, combine = _sc_calls()
    disp = dispatch(x, slot)
    y = _ffn(disp, w1, b1.reshape(E * NFC, 1, FCH), w2, b2.reshape(E, 1, D))
    out = combine(y, slot)
    return out.reshape(B, S, C), loss[0, 0]
